# baseline (speedup 1.0000x reference)
import functools

import jax
import jax.numpy as jnp
from jax import lax
from jax.experimental import pallas as pl
from jax.experimental.pallas import tpu as pltpu
from jax.experimental.pallas import tpu_sc as plsc

D_MODEL = 64
D_PAD = 128
SCALE_F = 8.0
NUM_CORES = 2
NUM_SUBCORES = 16
NUM_WORKERS = NUM_CORES * NUM_SUBCORES
LANES = 16
CHUNK = 400


def kernel(token_ids, table):
    batch_shape = token_ids.shape
    idx = token_ids.reshape(-1)
    num_ids = idx.shape[0]
    per_worker = num_ids // NUM_WORKERS
    n_chunks = per_worker // CHUNK
    assert per_worker * NUM_WORKERS == num_ids
    assert n_chunks * CHUNK == per_worker

    mesh = plsc.VectorSubcoreMesh(core_axis_name="c", subcore_axis_name="s")

    @functools.partial(
        pl.kernel,
        mesh=mesh,
        out_type=jax.ShapeDtypeStruct((num_ids, D_PAD), jnp.float32),
        scratch_types=[
            pltpu.VMEM((CHUNK,), jnp.int32),
            pltpu.VMEM((CHUNK, D_MODEL), jnp.float32),
            pltpu.VMEM((CHUNK, D_PAD), jnp.float32),
            pltpu.SemaphoreType.DMA,
        ],
        compiler_params=pltpu.CompilerParams(use_tc_tiling_on_sc=False),
    )
    def gather_scale(table_hbm, idx_hbm, out_hbm, idx_v, rows_v, padded_v,
                     sem):
        wid = lax.axis_index("s") * NUM_CORES + lax.axis_index("c")
        base0 = wid * per_worker

        @pl.loop(0, n_chunks)
        def _(j):
            base = base0 + j * CHUNK
            pltpu.sync_copy(idx_hbm.at[pl.ds(base, CHUNK)], idx_v)
            pltpu.async_copy(table_hbm.at[idx_v], rows_v, sem).wait()

            @pl.loop(0, CHUNK)
            def _(r):
                for c in range(0, D_MODEL, LANES):
                    padded_v.at[r, pl.ds(c, LANES)][...] = (
                        rows_v.at[r, pl.ds(c, LANES)][...] * SCALE_F
                    )

            pltpu.sync_copy(padded_v, out_hbm.at[pl.ds(base, CHUNK)])

    out = gather_scale(table, idx)
    return out[:, :D_MODEL].reshape(*batch_shape, D_MODEL)


# speedup vs baseline: 1.6259x; 1.6259x over previous
import functools

import jax
import jax.numpy as jnp
from jax import lax
from jax.experimental import pallas as pl
from jax.experimental.pallas import tpu as pltpu
from jax.experimental.pallas import tpu_sc as plsc

D_MODEL = 64
D_PAD = 128
SCALE_F = 8.0
NUM_CORES = 2
NUM_SUBCORES = 16
NUM_WORKERS = NUM_CORES * NUM_SUBCORES
LANES = 16
CHUNK = 800


def kernel(token_ids, table):
    batch_shape = token_ids.shape
    idx = token_ids.reshape(-1)
    num_ids = idx.shape[0]
    per_worker = num_ids // NUM_WORKERS
    n_chunks = per_worker // CHUNK
    assert per_worker * NUM_WORKERS == num_ids
    assert n_chunks * CHUNK == per_worker
    assert n_chunks >= 2

    mesh = plsc.VectorSubcoreMesh(core_axis_name="c", subcore_axis_name="s")

    @functools.partial(
        pl.kernel,
        mesh=mesh,
        out_type=jax.ShapeDtypeStruct((num_ids, D_PAD), jnp.float32),
        scratch_types=[
            pltpu.VMEM((CHUNK,), jnp.int32),
            pltpu.VMEM((CHUNK,), jnp.int32),
            pltpu.VMEM((CHUNK, D_MODEL), jnp.float32),
            pltpu.VMEM((CHUNK, D_MODEL), jnp.float32),
            pltpu.SemaphoreType.DMA,
            pltpu.SemaphoreType.DMA,
            pltpu.SemaphoreType.DMA,
            pltpu.SemaphoreType.DMA,
        ],
        compiler_params=pltpu.CompilerParams(use_tc_tiling_on_sc=False),
    )
    def gather_scale(table_hbm, idx_hbm, out_hbm, idx0, idx1, rows0, rows1,
                     sem_g0, sem_g1, sem_o0, sem_o1):
        wid = lax.axis_index("s") * NUM_CORES + lax.axis_index("c")
        base0 = wid * per_worker
        idx_v = (idx0, idx1)
        rows_v = (rows0, rows1)
        sem_g = (sem_g0, sem_g1)
        sem_o = (sem_o0, sem_o1)

        def start_gather(j, b):
            base = base0 + j * CHUNK
            pltpu.sync_copy(idx_hbm.at[pl.ds(base, CHUNK)], idx_v[b])
            return pltpu.async_copy(table_hbm.at[idx_v[b]], rows_v[b],
                                    sem_g[b])

        gathers = [None, None]
        outs = [None, None]
        gathers[0] = start_gather(0, 0)

        for j in range(n_chunks):
            b = j % 2
            if j + 1 < n_chunks:
                gathers[(j + 1) % 2] = start_gather(j + 1, (j + 1) % 2)
            gathers[b].wait()
            if outs[b] is not None:
                outs[b].wait()

            @pl.loop(0, CHUNK)
            def _(r):
                for c in range(0, D_MODEL, LANES):
                    sl = (r, pl.ds(c, LANES))
                    rows_v[b].at[sl][...] = rows_v[b].at[sl][...] * SCALE_F

            base = base0 + j * CHUNK
            outs[b] = pltpu.async_copy(
                rows_v[b],
                out_hbm.at[pl.ds(base, CHUNK), pl.ds(0, D_MODEL)],
                sem_o[b],
            )

        outs[(n_chunks - 2) % 2].wait()
        outs[(n_chunks - 1) % 2].wait()

    out = gather_scale(table, idx)
    return out[:, :D_MODEL].reshape(*batch_shape, D_MODEL)
